# Initial kernel scaffold; baseline (speedup 1.0000x reference)
import jax
import jax.numpy as jnp
from jax.experimental import pallas as pl


def kernel(inputs, word_table, pos_table):
    raise NotImplementedError("write your pallas kernel here")


# speedup vs baseline: 3.6210x; 3.6210x over previous
import functools
import jax
import jax.numpy as jnp
from jax import lax
from jax.experimental import pallas as pl
from jax.experimental.pallas import tpu as pltpu
from jax.experimental.pallas import tpu_sc as plsc

EMBED = 64
PAD = 128
SEQ = 200
CHUNKS = ((0, 128), (128, 72))
LANES = 16
NUM_WORKERS = 32


def kernel(inputs, word_table, pos_table):
    batch, seq = inputs.shape
    num_idx = batch * seq
    rows_per_w = batch // NUM_WORKERS
    idx_per_w = rows_per_w * seq
    flat_idx = inputs.reshape(num_idx)
    word_padded = jnp.pad(word_table, ((0, 0), (0, PAD - EMBED)))

    mesh = plsc.VectorSubcoreMesh(core_axis_name="c", subcore_axis_name="s")

    @functools.partial(
        pl.kernel,
        out_type=jax.ShapeDtypeStruct((num_idx, EMBED), jnp.float32),
        mesh=mesh,
        scratch_types=[
            pltpu.VMEM((idx_per_w,), jnp.int32),
            pltpu.VMEM((SEQ, EMBED), jnp.float32),
            pltpu.VMEM((CHUNKS[0][1], PAD), jnp.float32),
            pltpu.VMEM((SEQ, EMBED), jnp.float32),
        ],
    )
    def sc_kernel(word_hbm, idx_hbm, pos_hbm, out_hbm,
                  idx_v, pos_v, rows_v, stage_v):
        wid = lax.axis_index("s") * 2 + lax.axis_index("c")
        idx_base = pl.multiple_of(wid * idx_per_w, idx_per_w)
        pltpu.sync_copy(idx_hbm.at[pl.ds(idx_base, idx_per_w)], idx_v)
        pltpu.sync_copy(pos_hbm, pos_v)

        @pl.loop(0, rows_per_w)
        def _(t):
            t_base = pl.multiple_of(t * SEQ, SEQ)
            for start, size in CHUNKS:
                pltpu.sync_copy(
                    word_hbm.at[idx_v.at[pl.ds(t_base + start, size)]],
                    rows_v.at[pl.ds(0, size)],
                )

                @pl.loop(0, size)
                def _(r):
                    for c in range(0, EMBED, LANES):
                        stage_v.at[start + r, pl.ds(c, LANES)][...] = (
                            rows_v.at[r, pl.ds(c, LANES)][...]
                            + pos_v.at[start + r, pl.ds(c, LANES)][...]
                        )

            out_base = pl.multiple_of(idx_base + t_base, SEQ)
            pltpu.sync_copy(stage_v, out_hbm.at[pl.ds(out_base, SEQ)])

    out = sc_kernel(word_padded, flat_idx, pos_table)
    return out.reshape(batch, seq, EMBED)
